# Initial kernel scaffold; baseline (speedup 1.0000x reference)
import jax
import jax.numpy as jnp
from jax.experimental import pallas as pl


def kernel(H, W_lin, W_val, a, W_out, A0, src, dst):
    raise NotImplementedError("write your pallas kernel here")


# speedup vs baseline: 2.4877x; 2.4877x over previous
import functools
import jax
import jax.numpy as jnp
from jax import lax
from jax.experimental import pallas as pl
from jax.experimental.pallas import tpu as pltpu

N = 256
D = 128
HEADS = 4
DH = D // HEADS
BT = 64


def _mask_kernel(srcp_ref, dstp_ref, a0_ref, w_ref):
    ep = srcp_ref.shape[1]
    ii = lax.broadcasted_iota(jnp.int32, (N, ep), 0)
    p = (ii == srcp_ref[0][None, :]).astype(jnp.float32)
    jj = lax.broadcasted_iota(jnp.int32, (ep, N), 1)
    q = (jj == dstp_ref[:, 0][:, None]).astype(jnp.float32)
    cnt = jnp.dot(p, q, preferred_element_type=jnp.float32)
    w_ref[...] = cnt * (a0_ref[...] + 1e-8)


def _gat_kernel(h_ref, wl_ref, wv_ref, a_ref, wo_ref, wm_ref, out_ref):
    hb = h_ref[...]
    xq = lax.dot_general(hb, wl_ref[...], (((1,), (1,)), ((), ())),
                         preferred_element_type=jnp.float32)
    xv = lax.dot_general(hb, wv_ref[...], (((1,), (1,)), ((), ())),
                         preferred_element_type=jnp.float32)
    wm = wm_ref[...]
    edge = wm > 1e-9
    y_parts = []
    for h in range(HEADS):
        qh = xq[:, h * DH:(h + 1) * DH]
        qt = qh.T
        aq_col = jnp.zeros((N, 1), jnp.float32)
        for d in range(DH):
            aq_col = aq_col + a_ref[h, d] * qh[:, d:d + 1]
        aq_row = aq_col.T
        acc = 0.6 * (aq_col + aq_row)
        for d in range(DH):
            t = qh[:, d:d + 1] + qt[d:d + 1, :]
            acc = acc + (0.4 * a_ref[h, d]) * jnp.abs(t)
        e_mask = jnp.where(edge, acc, -1e30)
        m = jnp.max(e_mask, axis=1, keepdims=True)
        m = jnp.where(m <= -1e29, 0.0, m)
        num = wm * jnp.exp(e_mask - m)
        den = jnp.sum(num, axis=1, keepdims=True)
        attn = num / jnp.maximum(den, 1e-30)
        vh = xv[:, h * DH:(h + 1) * DH]
        y_parts.append(jnp.dot(attn, vh, preferred_element_type=jnp.float32))
    y = jnp.concatenate(y_parts, axis=1)
    out_ref[...] = lax.dot_general(y, wo_ref[...], (((1,), (1,)), ((), ())),
                                   preferred_element_type=jnp.float32)


def kernel(H, W_lin, W_val, a, W_out, A0, src, dst):
    e = src.shape[0]
    ep = ((e + 127) // 128) * 128
    srcp = jnp.full((1, ep), N, jnp.int32).at[0, :e].set(src)
    dstp = jnp.full((ep, 1), N, jnp.int32).at[:e, 0].set(dst)

    wmat = pl.pallas_call(
        _mask_kernel,
        out_shape=jax.ShapeDtypeStruct((N, N), jnp.float32),
    )(srcp, dstp, A0)

    out = pl.pallas_call(
        _gat_kernel,
        grid=(BT,),
        in_specs=[
            pl.BlockSpec((None, N, D), lambda i: (i, 0, 0)),
            pl.BlockSpec((D, D), lambda i: (0, 0)),
            pl.BlockSpec((D, D), lambda i: (0, 0)),
            pl.BlockSpec(memory_space=pltpu.SMEM),
            pl.BlockSpec((D, D), lambda i: (0, 0)),
            pl.BlockSpec((N, N), lambda i: (0, 0)),
        ],
        out_specs=pl.BlockSpec((None, N, D), lambda i: (i, 0, 0)),
        out_shape=jax.ShapeDtypeStruct((BT, N, D), jnp.float32),
    )(H, W_lin, W_val, a, W_out, wmat)
    return out


# speedup vs baseline: 3.4089x; 1.3703x over previous
import jax
import jax.numpy as jnp
from jax import lax
from jax.experimental import pallas as pl
from jax.experimental.pallas import tpu as pltpu
from jax.experimental.pallas import tpu_sc as plsc

N = 256
D = 128
HEADS = 4
DH = D // HEADS
BT = 64
ROW = D * BT
NC, NS, L = 2, 16, 16
NW = NC * NS
NPW = N // NW
CHUNK = 4
MAXDEG = 40
NBLK = 32


def _tc_proj_kernel(h_ref, wl_ref, wv_ref, xq_ref, xv_ref):
    for t in range(NBLK):
        hn = h_ref[:, t, :]
        xq_ref[t] = lax.dot_general(wl_ref[...], hn, (((1,), (1,)), ((), ())),
                                    preferred_element_type=jnp.float32)
        xv_ref[t] = lax.dot_general(wv_ref[...], hn, (((1,), (1,)), ((), ())),
                                    preferred_element_type=jnp.float32)


def _tc_out_kernel(y_ref, wo_ref, out_ref):
    for t in range(NBLK):
        out_ref[:, t, :] = lax.dot_general(
            y_ref[t], wo_ref[...], (((0,), (1,)), ((), ())),
            preferred_element_type=jnp.float32)


def _sc_edge_body(xq_hbm, xv_hbm, a0_hbm, ab_hbm, src_hbm, dst_hbm, y_hbm,
                  src_v, dst_v, ab_v, a0r, qrow, kbuf, ebuf, ybuf, sm):
    wid = lax.axis_index("s") * NC + lax.axis_index("c")
    base_node = wid * NPW
    epad = src_v.shape[0]
    lane0 = lax.iota(jnp.int32, 16) == 0
    zi16 = jnp.zeros((16,), jnp.int32)
    zf16v = jnp.zeros((16,), jnp.float32)

    def _lane0i(v):
        return jnp.max(jnp.where(lane0, v, zi16))

    def _lane0f(v):
        return jnp.max(jnp.where(lane0, v, zf16v))

    pltpu.sync_copy(src_hbm, src_v)
    pltpu.sync_copy(dst_hbm, dst_v)
    pltpu.sync_copy(ab_hbm, ab_v)
    pltpu.sync_copy(a0_hbm.at[pl.ds(base_node, NPW)], a0r)

    one16 = jnp.ones((16,), jnp.float32)
    zero16 = jnp.zeros((16,), jnp.float32)

    def cnt_body(i, carry):
        v = src_v[pl.ds(i * 16, 16)]
        c0 = carry[0] + jnp.where(v < base_node, one16, zero16)
        cts = tuple(carry[1 + t] + jnp.where(v == base_node + t, one16, zero16)
                    for t in range(NPW))
        return (c0,) + cts
    zeros_i = tuple(jnp.zeros((16,), jnp.float32) for _ in range(NPW + 1))
    carry = lax.fori_loop(0, epad // 16, cnt_body, zeros_i)
    run = jnp.sum(carry[0]).astype(jnp.int32)
    for t in range(NPW):
        sm[t] = run
        run = run + jnp.sum(carry[1 + t]).astype(jnp.int32)
    sm[NPW] = run

    def node_body(nt, _):
        est = sm[nt]
        deg = sm[nt + 1] - est
        node = base_node + nt
        pltpu.sync_copy(xq_hbm.at[pl.ds(node, 1)], qrow)

        def epass_body(eidx, _c):
            dnode = _lane0i(dst_v[pl.ds(est + eidx, 16)])
            pltpu.sync_copy(xq_hbm.at[pl.ds(dnode, 1)], kbuf)

            def d_body(d, acc):
                new = list(acc)
                for h in range(HEADS):
                    cvec = ab_v[h * DH + d]
                    for b in range(BT // 16):
                        off = (h * DH + d) * BT + 16 * b
                        q = qrow[0, pl.ds(off, 16)]
                        k = kbuf[0, pl.ds(off, 16)]
                        tt = q + k
                        u = jnp.maximum(tt, 0.2 * tt)
                        new[h * 4 + b] = new[h * 4 + b] + cvec * u
                return tuple(new)
            zf = tuple(jnp.zeros((16,), jnp.float32) for _ in range(16))
            acc = lax.fori_loop(0, DH, d_body, zf)
            for hb in range(16):
                ebuf[eidx, pl.ds(hb * 16, 16)] = acc[hb]
            return 0
        lax.fori_loop(0, deg, epass_body, 0)

        def m_body(e, m):
            return tuple(jnp.maximum(m[hb], ebuf[e, pl.ds(hb * 16, 16)])
                         for hb in range(16))
        minit = tuple(jnp.full((16,), -1e30, jnp.float32) for _ in range(16))
        m = lax.fori_loop(0, deg, m_body, minit)

        def s_body(e, den):
            dnode = _lane0i(dst_v[pl.ds(est + e, 16)])
            w = _lane0f(a0r[nt, pl.ds(dnode, 16)]) + 1e-8
            out = []
            for hb in range(16):
                x = w * jnp.exp(ebuf[e, pl.ds(hb * 16, 16)] - m[hb])
                ebuf[e, pl.ds(hb * 16, 16)] = x
                out.append(den[hb] + x)
            return tuple(out)
        zf16 = tuple(jnp.zeros((16,), jnp.float32) for _ in range(16))
        den = lax.fori_loop(0, deg, s_body, zf16)
        rden = [1.0 / jnp.maximum(den[hb], 1e-30) for hb in range(16)]

        def z_body(i, _c):
            ybuf[0, pl.ds(i * 16, 16)] = jnp.zeros((16,), jnp.float32)
            return 0
        lax.fori_loop(0, ROW // 16, z_body, 0)

        def ypass_body(eidx, _c):
            dnode = _lane0i(dst_v[pl.ds(est + eidx, 16)])
            pltpu.sync_copy(xv_hbm.at[pl.ds(dnode, 1)], kbuf)
            av = [ebuf[eidx, pl.ds(hb * 16, 16)] * rden[hb]
                  for hb in range(16)]

            def d_body(d, _d):
                for h in range(HEADS):
                    for b in range(BT // 16):
                        off = (h * DH + d) * BT + 16 * b
                        v = kbuf[0, pl.ds(off, 16)]
                        ybuf[0, pl.ds(off, 16)] = (
                            ybuf[0, pl.ds(off, 16)] + av[h * 4 + b] * v)
                return 0
            lax.fori_loop(0, DH, d_body, 0)
            return 0
        lax.fori_loop(0, deg, ypass_body, 0)
        pltpu.sync_copy(ybuf, y_hbm.at[pl.ds(node, 1)])
        return 0

    lax.fori_loop(0, NPW, node_body, 0)


def kernel(H, W_lin, W_val, a, W_out, A0, src, dst):
    e = src.shape[0]
    epad = ((e + 31) // 16) * 16
    srcp = jnp.full((epad,), 4 * N, jnp.int32).at[:e].set(src)
    dstp = jnp.zeros((epad,), jnp.int32).at[:e].set(dst)

    xqT, xvT = pl.pallas_call(
        _tc_proj_kernel,
        grid=(N // NBLK,),
        in_specs=[pl.BlockSpec((BT, NBLK, D), lambda i: (0, i, 0)),
                  pl.BlockSpec((D, D), lambda i: (0, 0)),
                  pl.BlockSpec((D, D), lambda i: (0, 0))],
        out_specs=[pl.BlockSpec((NBLK, D, BT), lambda i: (i, 0, 0)),
                   pl.BlockSpec((NBLK, D, BT), lambda i: (i, 0, 0))],
        out_shape=[jax.ShapeDtypeStruct((N, D, BT), jnp.float32),
                   jax.ShapeDtypeStruct((N, D, BT), jnp.float32)],
    )(H, W_lin, W_val)

    y_fn = pl.kernel(
        _sc_edge_body,
        out_type=jax.ShapeDtypeStruct((N, ROW), jnp.float32),
        mesh=plsc.VectorSubcoreMesh(core_axis_name="c", subcore_axis_name="s",
                                    num_cores=NC, num_subcores=NS),
        compiler_params=pltpu.CompilerParams(needs_layout_passes=False),
        scratch_types=[
            pltpu.VMEM((epad,), jnp.int32),
            pltpu.VMEM((epad,), jnp.int32),
            pltpu.VMEM((D, 16), jnp.float32),
            pltpu.VMEM((NPW, N + 16), jnp.float32),
            pltpu.VMEM((1, ROW), jnp.float32),
            pltpu.VMEM((1, ROW), jnp.float32),
            pltpu.VMEM((MAXDEG, 256), jnp.float32),
            pltpu.VMEM((1, ROW), jnp.float32),
            pltpu.SMEM((16,), jnp.int32),
        ],
    )
    abct = jnp.broadcast_to(a.reshape(D)[:, None], (D, 16))
    a0p = jnp.concatenate([A0, jnp.zeros((N, 16), jnp.float32)], axis=1)
    y = y_fn(xqT.reshape(N, ROW), xvT.reshape(N, ROW), a0p, abct, srcp, dstp)

    out = pl.pallas_call(
        _tc_out_kernel,
        grid=(N // NBLK,),
        in_specs=[pl.BlockSpec((NBLK, D, BT), lambda i: (i, 0, 0)),
                  pl.BlockSpec((D, D), lambda i: (0, 0))],
        out_specs=pl.BlockSpec((BT, NBLK, D), lambda i: (0, i, 0)),
        out_shape=jax.ShapeDtypeStruct((BT, N, D), jnp.float32),
    )(y.reshape(N, D, BT), W_out)
    return out
